# baseline (speedup 1.0000x reference)
import jax
import jax.numpy as jnp
from jax import lax
from jax.experimental import pallas as pl
from jax.experimental.pallas import tpu as pltpu
from jax.experimental.pallas import tpu_sc as plsc

VOCAB = 100000
D = 200
DP = 256
N_ROWS = 204800
NC, NS, L = 2, 16, 16
NW = NC * NS
PER_W = N_ROWS // NW
B = 4096
SEQ = 50
SEQP = 128
PER_WI = (B // NW) * SEQP
N_GROUPS = PER_WI // L


def _sc_body(idx_hbm, emb_hbm, out_hbm, idx_v, rows_v, rows2_v, sem):
    wid = lax.axis_index("s") * NC + lax.axis_index("c")
    sbase = wid * (B // NW)

    pltpu.sync_copy(idx_hbm.at[pl.ds(wid * PER_WI, PER_WI)], idx_v)

    vocab_v = jnp.full((L,), VOCAB, jnp.int32)

    def group_body(g, carry):
        v = idx_v[pl.ds(g * L, L)]
        idx_v[pl.ds(g * L, L)] = jnp.minimum(v, vocab_v)
        return carry

    lax.fori_loop(0, N_GROUPS, group_body, jnp.int32(0))

    def chunk_body(c, carry):
        cbase = c * (2 * SEQP)
        for j in range(2):
            pltpu.async_copy(
                emb_hbm.at[idx_v.at[pl.ds(cbase + j * SEQP, 56)]],
                rows_v.at[j], sem
            ).wait()
        def narrow_body(ln, cc):
            for sm in range(2):
                for d0 in range(12):
                    rows2_v[sm, ln, pl.ds(d0 * L, L)] = (
                        rows_v[sm, ln, pl.ds(d0 * L, L)])
                rows2_v[sm, ln, pl.ds(D - L, L)] = (
                    rows_v[sm, ln, pl.ds(D - L, L)])
            return cc

        lax.fori_loop(0, SEQ, narrow_body, jnp.int32(0))
        pltpu.sync_copy(rows2_v, out_hbm.at[pl.ds(sbase + c * 2, 2)])
        return carry

    lax.fori_loop(0, (B // NW) // 2, chunk_body, jnp.int32(0))


@jax.jit
def _run(idx_flat, table_p):
    mesh = plsc.VectorSubcoreMesh(core_axis_name="c", subcore_axis_name="s")
    f = pl.kernel(
        _sc_body,
        out_type=jax.ShapeDtypeStruct((B, SEQ, D), jnp.float32),
        mesh=mesh,
        scratch_types=[
            pltpu.VMEM((PER_WI,), jnp.int32),
            pltpu.VMEM((2, 56, DP), jnp.float32),
            pltpu.VMEM((2, SEQ, D), jnp.float32),
            pltpu.SemaphoreType.DMA,
        ],
    )
    return f(idx_flat, table_p)


def kernel(word_indices, embedding_matrix, oov_embedding):
    idx_flat = jnp.pad(
        word_indices.astype(jnp.int32), ((0, 0), (0, SEQP - SEQ))
    ).reshape(-1)
    table_p = jnp.pad(
        jnp.concatenate([embedding_matrix, oov_embedding], axis=0),
        ((0, 0), (0, DP - D)),
    )
    return _run(idx_flat, table_p)


# speedup vs baseline: 1.7859x; 1.7859x over previous
import jax
import jax.numpy as jnp
from jax import lax
from jax.experimental import pallas as pl
from jax.experimental.pallas import tpu as pltpu
from jax.experimental.pallas import tpu_sc as plsc

VOCAB = 100000
D = 200
DP = 256
N_ROWS = 204800
NC, NS, L = 2, 16, 16
NW = NC * NS
PER_W = N_ROWS // NW
CHUNK = 128
N_CHUNKS = PER_W // CHUNK
N_GROUPS = PER_W // L


def _sc_body(idx_hbm, emb_hbm, out_hbm, idx_v, rows_v, sem):
    wid = lax.axis_index("s") * NC + lax.axis_index("c")
    base = wid * PER_W

    pltpu.sync_copy(idx_hbm.at[pl.ds(base, PER_W)], idx_v)

    vocab_v = jnp.full((L,), VOCAB, jnp.int32)

    def group_body(g, carry):
        v = idx_v[pl.ds(g * L, L)]
        idx_v[pl.ds(g * L, L)] = jnp.minimum(v, vocab_v)
        return carry

    lax.fori_loop(0, N_GROUPS, group_body, jnp.int32(0))

    def chunk_body(c, carry):
        cbase = c * CHUNK
        pltpu.async_copy(
            emb_hbm.at[idx_v.at[pl.ds(cbase, CHUNK)]], rows_v, sem
        ).wait()
        pltpu.sync_copy(rows_v, out_hbm.at[pl.ds(base + cbase, CHUNK)])
        return carry

    lax.fori_loop(0, N_CHUNKS, chunk_body, jnp.int32(0))


@jax.jit
def _run(idx_flat, table_p):
    mesh = plsc.VectorSubcoreMesh(core_axis_name="c", subcore_axis_name="s")
    f = pl.kernel(
        _sc_body,
        out_type=jax.ShapeDtypeStruct((N_ROWS, DP), jnp.float32),
        mesh=mesh,
        scratch_types=[
            pltpu.VMEM((PER_W,), jnp.int32),
            pltpu.VMEM((CHUNK, DP), jnp.float32),
            pltpu.SemaphoreType.DMA,
        ],
    )
    return f(idx_flat, table_p)


def kernel(word_indices, embedding_matrix, oov_embedding):
    idx_flat = word_indices.reshape(-1).astype(jnp.int32)
    one = 1.0 + 0.0 * oov_embedding[0, 0]
    table_p = jnp.pad(
        jnp.concatenate([embedding_matrix, oov_embedding], axis=0),
        ((0, 0), (0, DP - D)),
    ) * one
    out = _run(idx_flat, table_p)
    return out[:, :D].reshape(word_indices.shape + (D,)) * one


# speedup vs baseline: 2.0172x; 1.1295x over previous
import jax
import jax.numpy as jnp
from jax import lax
from jax.experimental import pallas as pl
from jax.experimental.pallas import tpu as pltpu
from jax.experimental.pallas import tpu_sc as plsc

VOCAB = 100000
D = 200
DP = 256
N_ROWS = 204800
NC, NS, L = 2, 16, 16
NW = NC * NS
PER_W = N_ROWS // NW
CHUNK = 128
N_CHUNKS = PER_W // CHUNK
N_GROUPS = PER_W // L


def _sc_body(idx_hbm, emb_hbm, out_hbm, idx_v, rows_v, sem):
    wid = lax.axis_index("s") * NC + lax.axis_index("c")
    base = wid * PER_W

    pltpu.sync_copy(idx_hbm.at[pl.ds(base, PER_W)], idx_v)

    vocab_v = jnp.full((L,), VOCAB, jnp.int32)

    def group_body(g, carry):
        v = idx_v[pl.ds(g * L, L)]
        idx_v[pl.ds(g * L, L)] = jnp.minimum(v, vocab_v)
        return carry

    lax.fori_loop(0, N_GROUPS, group_body, jnp.int32(0))

    def chunk_body(c, carry):
        cbase = c * CHUNK
        pltpu.async_copy(
            emb_hbm.at[idx_v.at[pl.ds(cbase, CHUNK)]], rows_v, sem
        ).wait()
        pltpu.sync_copy(rows_v, out_hbm.at[pl.ds(base + cbase, CHUNK)])
        return carry

    lax.fori_loop(0, N_CHUNKS, chunk_body, jnp.int32(0))


@jax.jit
def _run(idx_flat, table_p):
    mesh = plsc.VectorSubcoreMesh(core_axis_name="c", subcore_axis_name="s")
    f = pl.kernel(
        _sc_body,
        out_type=jax.ShapeDtypeStruct((N_ROWS, DP), jnp.float32),
        mesh=mesh,
        scratch_types=[
            pltpu.VMEM((PER_W,), jnp.int32),
            pltpu.VMEM((CHUNK, DP), jnp.float32),
            pltpu.SemaphoreType.DMA,
        ],
    )
    return f(idx_flat, table_p)


def kernel(word_indices, embedding_matrix, oov_embedding):
    idx_flat = word_indices.reshape(-1).astype(jnp.int32)
    table_p = jnp.pad(
        jnp.concatenate([embedding_matrix, oov_embedding], axis=0),
        ((0, 0), (0, DP - D)),
    )
    out = _run(idx_flat, table_p)
    return out[:, :D].reshape(word_indices.shape + (D,))
